# baseline (speedup 1.0000x reference)
import functools
import math

import jax
import jax.numpy as jnp
from jax.experimental import pallas as pl
from jax.experimental.pallas import tpu as pltpu

HIDDEN = 128
NUM_LAYERS = 4
NUM_RBF = 50
CUTOFF = 5.0
N = 512
NMOL = 32
CHUNK = 32


def _dot(a, b, dims):
    return jax.lax.dot_general(a, b, (dims, ((), ())),
                               preferred_element_type=jnp.float32)


def _split(v):
    hi = v.astype(jnp.bfloat16).astype(jnp.float32)
    return hi, v - hi


def _dot3x(a, b, dims):
    a_hi, a_lo = _split(a)
    b_hi, b_lo = _split(b)
    return (_dot(a_hi, b_hi, dims) + _dot(a_hi, b_lo, dims)
            + _dot(a_lo, b_hi, dims))


def _silu(v):
    return v * jax.nn.sigmoid(v)


def _hdot(a, b, dims):
    return jax.lax.dot_general(a, b, (dims, ((), ())),
                               precision=jax.lax.Precision.HIGHEST,
                               preferred_element_type=jnp.float32)


def _fwd_kernel(an_ref, pos_ref, posT_ref, batch_ref, emb_ref, w1_ref, b1_ref, w2_ref,
                b2_ref, wih_ref, bih_ref, bhh_ref, lng_ref, lnb_ref,
                x_out_ref, g_out_ref, d_ref, mask_ref, a_ref):
    f32 = jnp.float32

    d2 = jnp.zeros((N, N), f32)
    for t in range(3):
        diff = pos_ref[:, t:t + 1] - posT_ref[t:t + 1, :]
        d2 = d2 + diff * diff
    d = jnp.sqrt(d2 + 1e-12)
    d_ref[...] = d
    notdiag = jax.lax.broadcasted_iota(jnp.int32, (N, N), 0) != \
        jax.lax.broadcasted_iota(jnp.int32, (N, N), 1)
    mask_ref[...] = jnp.where(notdiag & (d < CUTOFF), f32(1.0), f32(0.0))
    cnt = jnp.sum(mask_ref[...], axis=0)

    an = jnp.clip(an_ref[...], 0, 99)
    onehot = (an == jax.lax.broadcasted_iota(jnp.int32, (N, 100), 1)).astype(f32)
    x = _hdot(onehot, emb_ref[...], ((1,), (0,)))

    centers = jax.lax.broadcasted_iota(jnp.int32, (1, 1, NUM_RBF), 2) \
        .astype(f32) * (CUTOFF / (NUM_RBF - 1))
    inv2w2 = 1.0 / (2.0 * (CUTOFF / NUM_RBF) ** 2)

    nchunk = N // CHUNK
    for l in range(NUM_LAYERS):
        w1x = w1_ref[l, :, :HIDDEN]
        w1r = w1_ref[l, :, HIDDEN:]
        a_ref[...] = _dot(x, w1x, ((1,), (1,)))
        b1row = b1_ref[l][None, None, :]

        def body(c, s):
            dch = d_ref[pl.ds(c * CHUNK, CHUNK), :]
            mch = mask_ref[pl.ds(c * CHUNK, CHUNK), :]
            rbf = jnp.exp(-(dch[:, :, None] - centers) ** 2 * inv2w2)
            b = _dot(rbf, w1r, ((2,), (1,)))
            ach = a_ref[pl.ds(c * CHUNK, CHUNK), :]
            h = _silu((ach[:, None, :] + b) + b1row)
            h = h.astype(jnp.bfloat16).astype(f32)
            return s + jnp.sum(mch[:, :, None] * h, axis=0)

        s = jax.lax.fori_loop(0, nchunk, body,
                              jnp.zeros((N, HIDDEN), f32))
        s_hi, s_lo = _split(s)
        aggr = (_dot(s_hi, w2_ref[l], ((1,), (1,)))
                + _dot(s_lo, w2_ref[l], ((1,), (1,)))
                + cnt[:, None] * b2_ref[l][None, :])

        gi = _dot(aggr, wih_ref[l], ((1,), (1,))) + bih_ref[l][None, :]
        i_r = gi[:, :HIDDEN]
        i_z = gi[:, HIDDEN:2 * HIDDEN]
        i_n = gi[:, 2 * HIDDEN:]
        b_r = bhh_ref[l, :HIDDEN][None, :]
        b_z = bhh_ref[l, HIDDEN:2 * HIDDEN][None, :]
        b_n = bhh_ref[l, 2 * HIDDEN:][None, :]
        r = jax.nn.sigmoid(i_r + b_r)
        z = jax.nn.sigmoid(i_z + b_z)
        nact = jnp.tanh(i_n + r * b_n)
        x = x + (1.0 - z) * nact

    mu = jnp.mean(x, axis=-1, keepdims=True)
    var = jnp.mean((x - mu) ** 2, axis=-1, keepdims=True)
    x = (x - mu) / jnp.sqrt(var + 1e-5) * lng_ref[...][None, :] + \
        lnb_ref[...][None, :]
    x_out_ref[...] = x

    bsel = (batch_ref[...] ==
            jax.lax.broadcasted_iota(jnp.int32, (N, NMOL), 1)).astype(f32)
    gsum = _hdot(bsel, x, ((0,), (0,)))
    counts = jnp.sum(bsel, axis=0)
    g_out_ref[...] = gsum / jnp.maximum(counts, 1.0)[:, None]


@functools.partial(jax.jit, static_argnames=())
def _run(an, pos, posT, batch, emb, w1, b1, w2, b2, wih, bih, bhh, lng, lnb):
    return pl.pallas_call(
        _fwd_kernel,
        out_shape=(
            jax.ShapeDtypeStruct((N, HIDDEN), jnp.float32),
            jax.ShapeDtypeStruct((NMOL, HIDDEN), jnp.float32),
        ),
        scratch_shapes=[
            pltpu.VMEM((N, N), jnp.float32),
            pltpu.VMEM((N, N), jnp.float32),
            pltpu.VMEM((N, HIDDEN), jnp.float32),
        ],
    )(an, pos, posT, batch, emb, w1, b1, w2, b2, wih, bih, bhh, lng, lnb)


def kernel(atomic_numbers, positions, batch, emb, msg_w1, msg_b1, msg_w2,
           msg_b2, gru_wih, gru_bih, gru_whh, gru_bhh, ln_g, ln_b):
    an = atomic_numbers.astype(jnp.int32).reshape(N, 1)
    bt = batch.astype(jnp.int32).reshape(N, 1)
    x, g = _run(an, positions, positions.T, bt, emb, msg_w1, msg_b1, msg_w2, msg_b2,
                gru_wih, gru_bih, gru_bhh, ln_g, ln_b)
    return x, g


# speedup vs baseline: 1.0084x; 1.0084x over previous
import functools

import jax
import jax.numpy as jnp
from jax.experimental import pallas as pl
from jax.experimental.pallas import tpu as pltpu

HIDDEN = 128
NUM_LAYERS = 4
NUM_RBF = 50
CUTOFF = 5.0
N = 512
NMOL = 32
CHUNK = 64


def _dot(a, b, dims):
    return jax.lax.dot_general(a, b, (dims, ((), ())),
                               preferred_element_type=jnp.float32)


def _split(v):
    hi = v.astype(jnp.bfloat16).astype(jnp.float32)
    return hi, v - hi


def _silu(v):
    return v * jax.nn.sigmoid(v)


def _hdot(a, b, dims):
    return jax.lax.dot_general(a, b, (dims, ((), ())),
                               precision=jax.lax.Precision.HIGHEST,
                               preferred_element_type=jnp.float32)


def _fwd_kernel(an_ref, pos_ref, posT_ref, batch_ref, emb_ref, w1_ref, b1_ref, w2_ref,
                b2_ref, wih_ref, bih_ref, bhh_ref, lng_ref, lnb_ref,
                x_out_ref, g_out_ref, d_ref, mask_ref, a_ref):
    f32 = jnp.float32

    d2 = jnp.zeros((N, N), f32)
    for t in range(3):
        diff = pos_ref[:, t:t + 1] - posT_ref[t:t + 1, :]
        d2 = d2 + diff * diff
    d = jnp.sqrt(d2 + 1e-12)
    d_ref[...] = d
    notdiag = jax.lax.broadcasted_iota(jnp.int32, (N, N), 0) != \
        jax.lax.broadcasted_iota(jnp.int32, (N, N), 1)
    mask_ref[...] = jnp.where(notdiag & (d < CUTOFF), f32(1.0), f32(0.0))
    cnt = jnp.sum(mask_ref[...], axis=0)

    an = jnp.clip(an_ref[...], 0, 99)
    onehot = (an == jax.lax.broadcasted_iota(jnp.int32, (N, 100), 1)).astype(f32)
    x = _hdot(onehot, emb_ref[...], ((1,), (0,)))

    centers = jax.lax.broadcasted_iota(jnp.int32, (1, 1, NUM_RBF), 2) \
        .astype(f32) * (CUTOFF / (NUM_RBF - 1))
    inv2w2 = 1.0 / (2.0 * (CUTOFF / NUM_RBF) ** 2)

    nchunk = N // CHUNK
    for l in range(NUM_LAYERS):
        w1x = w1_ref[l, :, :HIDDEN]
        w1r = w1_ref[l, :, HIDDEN:]
        a_ref[...] = _dot(x, w1x, ((1,), (1,))) + b1_ref[l][None, :]

        def body(c, s):
            dch = d_ref[pl.ds(c * CHUNK, CHUNK), :]
            mch = mask_ref[pl.ds(c * CHUNK, CHUNK), :]
            rbf = jnp.exp(-(dch[:, :, None] - centers) ** 2 * inv2w2)
            b = _dot(rbf, w1r, ((2,), (1,)))
            ach = a_ref[pl.ds(c * CHUNK, CHUNK), :]
            h = _silu(ach[:, None, :] + b)
            h = h.astype(jnp.bfloat16).astype(f32)
            return s + jnp.sum(mch[:, :, None] * h, axis=0)

        s = jax.lax.fori_loop(0, nchunk, body,
                              jnp.zeros((N, HIDDEN), f32))
        s_hi, s_lo = _split(s)
        aggr = (_dot(s_hi, w2_ref[l], ((1,), (1,)))
                + _dot(s_lo, w2_ref[l], ((1,), (1,)))
                + cnt[:, None] * b2_ref[l][None, :])

        gi = _dot(aggr, wih_ref[l], ((1,), (1,))) + bih_ref[l][None, :]
        i_r = gi[:, :HIDDEN]
        i_z = gi[:, HIDDEN:2 * HIDDEN]
        i_n = gi[:, 2 * HIDDEN:]
        b_r = bhh_ref[l, :HIDDEN][None, :]
        b_z = bhh_ref[l, HIDDEN:2 * HIDDEN][None, :]
        b_n = bhh_ref[l, 2 * HIDDEN:][None, :]
        r = jax.nn.sigmoid(i_r + b_r)
        z = jax.nn.sigmoid(i_z + b_z)
        nact = jnp.tanh(i_n + r * b_n)
        x = x + (1.0 - z) * nact

    mu = jnp.mean(x, axis=-1, keepdims=True)
    var = jnp.mean((x - mu) ** 2, axis=-1, keepdims=True)
    x = (x - mu) / jnp.sqrt(var + 1e-5) * lng_ref[...][None, :] + \
        lnb_ref[...][None, :]
    x_out_ref[...] = x

    bsel = (batch_ref[...] ==
            jax.lax.broadcasted_iota(jnp.int32, (N, NMOL), 1)).astype(f32)
    gsum = _hdot(bsel, x, ((0,), (0,)))
    counts = jnp.sum(bsel, axis=0)
    g_out_ref[...] = gsum / jnp.maximum(counts, 1.0)[:, None]


@functools.partial(jax.jit, static_argnames=())
def _run(an, pos, posT, batch, emb, w1, b1, w2, b2, wih, bih, bhh, lng, lnb):
    return pl.pallas_call(
        _fwd_kernel,
        out_shape=(
            jax.ShapeDtypeStruct((N, HIDDEN), jnp.float32),
            jax.ShapeDtypeStruct((NMOL, HIDDEN), jnp.float32),
        ),
        scratch_shapes=[
            pltpu.VMEM((N, N), jnp.float32),
            pltpu.VMEM((N, N), jnp.float32),
            pltpu.VMEM((N, HIDDEN), jnp.float32),
        ],
    )(an, pos, posT, batch, emb, w1, b1, w2, b2, wih, bih, bhh, lng, lnb)


def kernel(atomic_numbers, positions, batch, emb, msg_w1, msg_b1, msg_w2,
           msg_b2, gru_wih, gru_bih, gru_whh, gru_bhh, ln_g, ln_b):
    an = atomic_numbers.astype(jnp.int32).reshape(N, 1)
    bt = batch.astype(jnp.int32).reshape(N, 1)
    x, g = _run(an, positions, positions.T, bt, emb, msg_w1, msg_b1, msg_w2, msg_b2,
                gru_wih, gru_bih, gru_bhh, ln_g, ln_b)
    return x, g
